# baseline (speedup 1.0000x reference)
import jax
import jax.numpy as jnp
from jax.experimental import pallas as pl
from jax.experimental.pallas import tpu as pltpu

_R = 4


def _dup_linear_kernel(x_ref, w_ref, b_ref, o_ref):
    y = jax.lax.dot_general(
        x_ref[...], w_ref[...], (((1,), (1,)), ((), ())),
        preferred_element_type=jnp.float32)
    y = y + b_ref[...]
    for r in range(_R):
        o_ref[r::_R, :] = y


def kernel(x, edge_index, W, b):
    n, c_in = x.shape
    c_out = W.shape[0]
    b2 = b.reshape(1, c_out)

    bn = 10000
    grid = (n // bn,)
    out = pl.pallas_call(
        _dup_linear_kernel,
        grid=grid,
        in_specs=[
            pl.BlockSpec((bn, c_in), lambda i: (i, i - i)),
            pl.BlockSpec((c_out, c_in), lambda i: (i - i, i - i)),
            pl.BlockSpec((1, c_out), lambda i: (i - i, i - i)),
        ],
        out_specs=pl.BlockSpec((_R * bn, c_out), lambda i: (i, i - i)),
        out_shape=jax.ShapeDtypeStruct((_R * n, c_out), jnp.float32),
        compiler_params=pltpu.CompilerParams(
            dimension_semantics=("parallel",)),
    )(x, W, b2)
    return out
